# Initial kernel scaffold; baseline (speedup 1.0000x reference)
import jax
import jax.numpy as jnp
from jax.experimental import pallas as pl


def kernel(features, edge_index, W1_self, W1_neigh, b1, W2_self, W2_neigh, b2):
    raise NotImplementedError("write your pallas kernel here")


# speedup vs baseline: 7.2978x; 7.2978x over previous
import functools

import jax
import jax.numpy as jnp
from jax import lax
from jax.experimental import pallas as pl
from jax.experimental.pallas import tpu as pltpu
from jax.experimental.pallas import tpu_sc as plsc

N = 10000
E = 320000
D_IN = 128
D_HID = 256
D_OUT = 128
AUG = 144

NC = 2
NS = 16
NW = NC * NS

CHUNK = 128
EPW = E // NW
FULL = EPW // CHUNK
TAIL = EPW - FULL * CHUNK
N_PAD = 10240
ROWS_PER_TILE = N_PAD // NS


def _make_sc_agg(d):
    mesh = plsc.VectorSubcoreMesh(core_axis_name="c", subcore_axis_name="s")

    @functools.partial(
        pl.kernel,
        mesh=mesh,
        compiler_params=pltpu.CompilerParams(use_tc_tiling_on_sc=False),
        out_type=jax.ShapeDtypeStruct((NC, N_PAD, d), jnp.float32),
        scratch_types=[
            pltpu.VMEM_SHARED((N_PAD, d), jnp.float32),
            pltpu.VMEM((CHUNK,), jnp.int32),
            pltpu.VMEM((CHUNK,), jnp.int32),
            pltpu.VMEM((CHUNK, d), jnp.float32),
            pltpu.VMEM((TAIL,), jnp.int32),
            pltpu.VMEM((TAIL,), jnp.int32),
            pltpu.VMEM((TAIL, d), jnp.float32),
            pltpu.SemaphoreType.DMA,
        ],
    )
    def k(table_hbm, src_hbm, dst_hbm, zeros_hbm, out_hbm,
          acc, src_v, dst_v, rows_v, tsrc_v, tdst_v, trows_v, sem):
        c = lax.axis_index("c")
        s = lax.axis_index("s")
        r0 = s * ROWS_PER_TILE
        pltpu.sync_copy(zeros_hbm.at[pl.ds(r0, ROWS_PER_TILE)],
                        acc.at[pl.ds(r0, ROWS_PER_TILE)])
        plsc.subcore_barrier()

        base = (c * NS + s) * EPW

        def body(i, carry):
            off = base + i * CHUNK
            pltpu.sync_copy(src_hbm.at[pl.ds(off, CHUNK)], src_v)
            pltpu.sync_copy(dst_hbm.at[pl.ds(off, CHUNK)], dst_v)
            pltpu.async_copy(table_hbm.at[src_v], rows_v, sem).wait()
            pltpu.sync_copy(rows_v, acc.at[dst_v], add=True)
            return carry

        lax.fori_loop(0, FULL, body, 0)

        toff = base + FULL * CHUNK
        pltpu.sync_copy(src_hbm.at[pl.ds(toff, TAIL)], tsrc_v)
        pltpu.sync_copy(dst_hbm.at[pl.ds(toff, TAIL)], tdst_v)
        pltpu.async_copy(table_hbm.at[tsrc_v], trows_v, sem).wait()
        pltpu.sync_copy(trows_v, acc.at[tdst_v], add=True)

        plsc.subcore_barrier()
        pltpu.sync_copy(acc.at[pl.ds(r0, ROWS_PER_TILE)],
                        out_hbm.at[c, pl.ds(r0, ROWS_PER_TILE)])

    return k


_sc_agg_aug = _make_sc_agg(AUG)
_sc_agg_128 = _make_sc_agg(D_OUT)

BLK = 1000


def _tc1_body(p_ref, f_ref, w1s_ref, w1n_ref, b1_ref, w2s_ref, w2n_ref,
              b2_ref, u2_ref, v2_ref, invd_ref):
    p = p_ref[0] + p_ref[1]
    deg = jnp.sum(p[:, 128:144], axis=1, keepdims=True) * (1.0 / 16.0)
    invdeg = 1.0 / jnp.maximum(deg, 1.0)
    hn = p[:, :128] * invdeg
    h1 = f_ref[...] @ w1s_ref[...] + hn @ w1n_ref[...] + b1_ref[...]
    h1 = jnp.maximum(h1, 0.0)
    u2_ref[...] = h1 @ w2n_ref[...]
    v2_ref[...] = h1 @ w2s_ref[...] + b2_ref[...]
    invd_ref[...] = jnp.broadcast_to(invdeg, (BLK, D_OUT))


def _tc1(p1, features, w1s, w1n, b1, w2s, w2n, b2):
    grid = (N // BLK,)
    return pl.pallas_call(
        _tc1_body,
        grid=grid,
        in_specs=[
            pl.BlockSpec((NC, BLK, AUG), lambda i: (0, i, 0)),
            pl.BlockSpec((BLK, D_IN), lambda i: (i, 0)),
            pl.BlockSpec((D_IN, D_HID), lambda i: (0, 0)),
            pl.BlockSpec((D_IN, D_HID), lambda i: (0, 0)),
            pl.BlockSpec((1, D_HID), lambda i: (0, 0)),
            pl.BlockSpec((D_HID, D_OUT), lambda i: (0, 0)),
            pl.BlockSpec((D_HID, D_OUT), lambda i: (0, 0)),
            pl.BlockSpec((1, D_OUT), lambda i: (0, 0)),
        ],
        out_specs=[
            pl.BlockSpec((BLK, D_OUT), lambda i: (i, 0)),
            pl.BlockSpec((BLK, D_OUT), lambda i: (i, 0)),
            pl.BlockSpec((BLK, D_OUT), lambda i: (i, 0)),
        ],
        out_shape=[
            jax.ShapeDtypeStruct((N, D_OUT), jnp.float32),
            jax.ShapeDtypeStruct((N, D_OUT), jnp.float32),
            jax.ShapeDtypeStruct((N, D_OUT), jnp.float32),
        ],
    )(p1, features, w1s, w1n, b1, w2s, w2n, b2)


def _tc2_body(q_ref, v2_ref, invd_ref, out_ref):
    out_ref[...] = v2_ref[...] + (q_ref[0] + q_ref[1]) * invd_ref[...]


def _tc2(p2, v2, invd):
    grid = (N // BLK,)
    return pl.pallas_call(
        _tc2_body,
        grid=grid,
        in_specs=[
            pl.BlockSpec((NC, BLK, D_OUT), lambda i: (0, i, 0)),
            pl.BlockSpec((BLK, D_OUT), lambda i: (i, 0)),
            pl.BlockSpec((BLK, D_OUT), lambda i: (i, 0)),
        ],
        out_specs=pl.BlockSpec((BLK, D_OUT), lambda i: (i, 0)),
        out_shape=jax.ShapeDtypeStruct((N, D_OUT), jnp.float32),
    )(p2, v2, invd)


def kernel(features, edge_index, W1_self, W1_neigh, b1, W2_self, W2_neigh, b2):
    src = edge_index[0]
    dst = edge_index[1]
    feat_aug = jnp.concatenate(
        [features, jnp.ones((N, AUG - D_IN), jnp.float32)], axis=1)
    zeros_aug = jnp.zeros((N_PAD, AUG), jnp.float32)
    zeros_out = jnp.zeros((N_PAD, D_OUT), jnp.float32)

    p1 = _sc_agg_aug(feat_aug, src, dst, zeros_aug)
    u2, v2, invd = _tc1(p1, features, W1_self, W1_neigh,
                        b1.reshape(1, D_HID), W2_self, W2_neigh,
                        b2.reshape(1, D_OUT))
    p2 = _sc_agg_128(u2, src, dst, zeros_out)
    return _tc2(p2, v2, invd)


# speedup vs baseline: 12.0064x; 1.6452x over previous
import functools

import jax
import jax.numpy as jnp
from jax import lax
from jax.experimental import pallas as pl
from jax.experimental.pallas import tpu as pltpu
from jax.experimental.pallas import tpu_sc as plsc

N = 10000
E = 320000
D_IN = 128
D_HID = 256
D_OUT = 128
AUG = 144

NC = 2
NS = 16
NW = NC * NS

CHUNK = 64
NCHUNK = E // CHUNK
CPW = NCHUNK // NW
CPW_LAST = NCHUNK - CPW * (NW - 1)
N_PAD = 10112
ROWS_PER_TILE = N_PAD // NS


def _make_sc_agg(d):
    mesh = plsc.VectorSubcoreMesh(core_axis_name="c", subcore_axis_name="s")

    @functools.partial(
        pl.kernel,
        mesh=mesh,
        compiler_params=pltpu.CompilerParams(use_tc_tiling_on_sc=False),
        out_type=jax.ShapeDtypeStruct((NC, N_PAD, d), jnp.float32),
        scratch_types=[
            pltpu.VMEM_SHARED((N_PAD, d), jnp.float32),
            pltpu.VMEM((CPW_LAST, CHUNK), jnp.int32),
            pltpu.VMEM((CPW_LAST, CHUNK), jnp.int32),
            pltpu.VMEM((CHUNK, d), jnp.float32),
            pltpu.VMEM((CHUNK, d), jnp.float32),
            pltpu.SemaphoreType.DMA,
            pltpu.SemaphoreType.DMA,
        ],
    )
    def k(table_hbm, src2d_hbm, dst2d_hbm, zeros_hbm, out_hbm,
          acc, src_all, dst_all, rows0, rows1, sem0, sem1):
        c = lax.axis_index("c")
        s = lax.axis_index("s")
        wid = c * NS + s
        r0 = s * ROWS_PER_TILE
        row0 = wid * CPW
        pltpu.sync_copy(src2d_hbm.at[pl.ds(row0, CPW_LAST)], src_all)
        pltpu.sync_copy(dst2d_hbm.at[pl.ds(row0, CPW_LAST)], dst_all)
        pltpu.sync_copy(zeros_hbm.at[pl.ds(r0, ROWS_PER_TILE)],
                        acc.at[pl.ds(r0, ROWS_PER_TILE)])
        nchunks = jnp.where(wid == NW - 1, CPW_LAST, CPW)
        pltpu.async_copy(table_hbm.at[src_all.at[0]], rows0, sem0)
        pltpu.async_copy(table_hbm.at[src_all.at[1]], rows1, sem1)
        plsc.subcore_barrier()

        def pair(g, carry):
            for b, rows, sem in ((0, rows0, sem0), (1, rows1, sem1)):
                i = 2 * g + b
                pltpu.make_async_copy(
                    table_hbm.at[pl.ds(0, CHUNK)], rows, sem).wait()
                pltpu.sync_copy(rows, acc.at[dst_all.at[i]], add=True)
                j = jnp.minimum(i + 2, nchunks - 1)
                pltpu.async_copy(table_hbm.at[src_all.at[j]], rows, sem)
            return carry

        lax.fori_loop(0, nchunks // 2, pair, 0)
        pltpu.make_async_copy(table_hbm.at[pl.ds(0, CHUNK)], rows0, sem0).wait()
        pltpu.make_async_copy(table_hbm.at[pl.ds(0, CHUNK)], rows1, sem1).wait()

        plsc.subcore_barrier()
        pltpu.sync_copy(acc.at[pl.ds(r0, ROWS_PER_TILE)],
                        out_hbm.at[c, pl.ds(r0, ROWS_PER_TILE)])

    return k


_sc_agg_aug = _make_sc_agg(AUG)
_sc_agg_128 = _make_sc_agg(D_OUT)

BLK = 1000


def _tc1_body(p_ref, f_ref, w1s_ref, w1n_ref, b1_ref, w2s_ref, w2n_ref,
              b2_ref, u2_ref, v2_ref, invd_ref):
    p = p_ref[0] + p_ref[1]
    deg = jnp.sum(p[:, 128:144], axis=1, keepdims=True) * (1.0 / 16.0)
    invdeg = 1.0 / jnp.maximum(deg, 1.0)
    hn = p[:, :128] * invdeg
    h1 = f_ref[...] @ w1s_ref[...] + hn @ w1n_ref[...] + b1_ref[...]
    h1 = jnp.maximum(h1, 0.0)
    u2_ref[...] = h1 @ w2n_ref[...]
    v2_ref[...] = h1 @ w2s_ref[...] + b2_ref[...]
    invd_ref[...] = jnp.broadcast_to(invdeg, (BLK, D_OUT))


def _tc1(p1, features, w1s, w1n, b1, w2s, w2n, b2):
    grid = (N // BLK,)
    return pl.pallas_call(
        _tc1_body,
        grid=grid,
        in_specs=[
            pl.BlockSpec((NC, BLK, AUG), lambda i: (0, i, 0)),
            pl.BlockSpec((BLK, D_IN), lambda i: (i, 0)),
            pl.BlockSpec((D_IN, D_HID), lambda i: (0, 0)),
            pl.BlockSpec((D_IN, D_HID), lambda i: (0, 0)),
            pl.BlockSpec((1, D_HID), lambda i: (0, 0)),
            pl.BlockSpec((D_HID, D_OUT), lambda i: (0, 0)),
            pl.BlockSpec((D_HID, D_OUT), lambda i: (0, 0)),
            pl.BlockSpec((1, D_OUT), lambda i: (0, 0)),
        ],
        out_specs=[
            pl.BlockSpec((BLK, D_OUT), lambda i: (i, 0)),
            pl.BlockSpec((BLK, D_OUT), lambda i: (i, 0)),
            pl.BlockSpec((BLK, D_OUT), lambda i: (i, 0)),
        ],
        out_shape=[
            jax.ShapeDtypeStruct((N, D_OUT), jnp.float32),
            jax.ShapeDtypeStruct((N, D_OUT), jnp.float32),
            jax.ShapeDtypeStruct((N, D_OUT), jnp.float32),
        ],
    )(p1, features, w1s, w1n, b1, w2s, w2n, b2)


def _tc2_body(q_ref, v2_ref, invd_ref, out_ref):
    out_ref[...] = v2_ref[...] + (q_ref[0] + q_ref[1]) * invd_ref[...]


def _tc2(p2, v2, invd):
    grid = (N // BLK,)
    return pl.pallas_call(
        _tc2_body,
        grid=grid,
        in_specs=[
            pl.BlockSpec((NC, BLK, D_OUT), lambda i: (0, i, 0)),
            pl.BlockSpec((BLK, D_OUT), lambda i: (i, 0)),
            pl.BlockSpec((BLK, D_OUT), lambda i: (i, 0)),
        ],
        out_specs=pl.BlockSpec((BLK, D_OUT), lambda i: (i, 0)),
        out_shape=jax.ShapeDtypeStruct((N, D_OUT), jnp.float32),
    )(p2, v2, invd)


def kernel(features, edge_index, W1_self, W1_neigh, b1, W2_self, W2_neigh, b2):
    src = edge_index[0].reshape(NCHUNK, CHUNK)
    dst = edge_index[1].reshape(NCHUNK, CHUNK)
    feat_aug = jnp.concatenate(
        [features, jnp.ones((N, AUG - D_IN), jnp.float32)], axis=1)
    zeros_aug = jnp.zeros((N_PAD, AUG), jnp.float32)
    zeros_out = jnp.zeros((N_PAD, D_OUT), jnp.float32)

    p1 = _sc_agg_aug(feat_aug, src, dst, zeros_aug)
    u2, v2, invd = _tc1(p1, features, W1_self, W1_neigh,
                        b1.reshape(1, D_HID), W2_self, W2_neigh,
                        b2.reshape(1, D_OUT))
    p2 = _sc_agg_128(u2, src, dst, zeros_out)
    return _tc2(p2, v2, invd)


# speedup vs baseline: 12.3571x; 1.0292x over previous
import functools

import jax
import jax.numpy as jnp
from jax import lax
from jax.experimental import pallas as pl
from jax.experimental.pallas import tpu as pltpu
from jax.experimental.pallas import tpu_sc as plsc

N = 10000
E = 320000
D_IN = 128
D_HID = 256
D_OUT = 128
AUG = 144

NC = 2
NS = 16
NW = NC * NS

CHUNK = 64
NCHUNK = E // CHUNK
CPW = NCHUNK // NW
CPW_LAST = NCHUNK - CPW * (NW - 1)
N_PAD = 10112
ROWS_PER_TILE = N_PAD // NS


def _make_sc_agg(d):
    mesh = plsc.VectorSubcoreMesh(core_axis_name="c", subcore_axis_name="s")

    @functools.partial(
        pl.kernel,
        mesh=mesh,
        compiler_params=pltpu.CompilerParams(use_tc_tiling_on_sc=False),
        out_type=jax.ShapeDtypeStruct((NC, N_PAD, d), jnp.float32),
        scratch_types=[
            pltpu.VMEM_SHARED((N_PAD, d), jnp.float32),
            pltpu.VMEM((CPW_LAST, CHUNK), jnp.int32),
            pltpu.VMEM((CPW_LAST, CHUNK), jnp.int32),
            pltpu.VMEM((CHUNK, d), jnp.float32),
            pltpu.VMEM((CHUNK, d), jnp.float32),
            pltpu.SemaphoreType.DMA,
            pltpu.SemaphoreType.DMA,
        ],
    )
    def k(table_hbm, edges_hbm, zeros_hbm, out_hbm,
          acc, src_all, dst_all, rows0, rows1, sem0, sem1):
        c = lax.axis_index("c")
        s = lax.axis_index("s")
        wid = c * NS + s
        r0 = s * ROWS_PER_TILE
        row0 = wid * CPW
        pltpu.sync_copy(edges_hbm.at[0, pl.ds(row0, CPW_LAST)], src_all)
        pltpu.sync_copy(edges_hbm.at[1, pl.ds(row0, CPW_LAST)], dst_all)
        nchunks = jnp.where(wid == NW - 1, CPW_LAST, CPW)
        pltpu.async_copy(table_hbm.at[src_all.at[0]], rows0, sem0)
        pltpu.async_copy(table_hbm.at[src_all.at[1]], rows1, sem1)
        pltpu.sync_copy(zeros_hbm.at[pl.ds(r0, ROWS_PER_TILE)],
                        acc.at[pl.ds(r0, ROWS_PER_TILE)])
        plsc.subcore_barrier()

        def pair(g, carry):
            for b, rows, sem in ((0, rows0, sem0), (1, rows1, sem1)):
                i = 2 * g + b
                pltpu.make_async_copy(
                    table_hbm.at[pl.ds(0, CHUNK)], rows, sem).wait()
                pltpu.sync_copy(rows, acc.at[dst_all.at[i]], add=True)
                j = jnp.minimum(i + 2, nchunks - 1)
                pltpu.async_copy(table_hbm.at[src_all.at[j]], rows, sem)
            return carry

        lax.fori_loop(0, nchunks // 2, pair, 0)
        pltpu.make_async_copy(table_hbm.at[pl.ds(0, CHUNK)], rows0, sem0).wait()
        pltpu.make_async_copy(table_hbm.at[pl.ds(0, CHUNK)], rows1, sem1).wait()

        plsc.subcore_barrier()
        pltpu.sync_copy(acc.at[pl.ds(r0, ROWS_PER_TILE)],
                        out_hbm.at[c, pl.ds(r0, ROWS_PER_TILE)])

    return k


_sc_agg_aug = _make_sc_agg(AUG)
_sc_agg_128 = _make_sc_agg(D_OUT)

BLK = 1000


def _tc1_body(p_ref, f_ref, w1s_ref, w1n_ref, b1_ref, w2s_ref, w2n_ref,
              b2_ref, u2_ref, v2_ref, invd_ref):
    p = p_ref[0] + p_ref[1]
    deg = jnp.sum(p[:, 128:144], axis=1, keepdims=True) * (1.0 / 16.0)
    invdeg = 1.0 / jnp.maximum(deg, 1.0)
    hn = p[:, :128] * invdeg
    h1 = f_ref[...] @ w1s_ref[...] + hn @ w1n_ref[...] + b1_ref[...]
    h1 = jnp.maximum(h1, 0.0)
    u2_ref[...] = h1 @ w2n_ref[...]
    v2_ref[...] = h1 @ w2s_ref[...] + b2_ref[...]
    invd_ref[...] = jnp.broadcast_to(invdeg, (BLK, 8))


def _tc1(p1, features, w1s, w1n, b1, w2s, w2n, b2):
    grid = (N // BLK,)
    return pl.pallas_call(
        _tc1_body,
        grid=grid,
        in_specs=[
            pl.BlockSpec((NC, BLK, AUG), lambda i: (0, i, 0)),
            pl.BlockSpec((BLK, D_IN), lambda i: (i, 0)),
            pl.BlockSpec((D_IN, D_HID), lambda i: (0, 0)),
            pl.BlockSpec((D_IN, D_HID), lambda i: (0, 0)),
            pl.BlockSpec((1, D_HID), lambda i: (0, 0)),
            pl.BlockSpec((D_HID, D_OUT), lambda i: (0, 0)),
            pl.BlockSpec((D_HID, D_OUT), lambda i: (0, 0)),
            pl.BlockSpec((1, D_OUT), lambda i: (0, 0)),
        ],
        out_specs=[
            pl.BlockSpec((BLK, D_OUT), lambda i: (i, 0)),
            pl.BlockSpec((BLK, D_OUT), lambda i: (i, 0)),
            pl.BlockSpec((BLK, 8), lambda i: (i, 0)),
        ],
        out_shape=[
            jax.ShapeDtypeStruct((N, D_OUT), jnp.float32),
            jax.ShapeDtypeStruct((N, D_OUT), jnp.float32),
            jax.ShapeDtypeStruct((N, 8), jnp.float32),
        ],
    )(p1, features, w1s, w1n, b1, w2s, w2n, b2)


def _tc2_body(q_ref, v2_ref, invd_ref, out_ref):
    invd = jnp.broadcast_to(invd_ref[...][:, :1], (BLK, D_OUT))
    out_ref[...] = v2_ref[...] + (q_ref[0] + q_ref[1]) * invd


def _tc2(p2, v2, invd):
    grid = (N // BLK,)
    return pl.pallas_call(
        _tc2_body,
        grid=grid,
        in_specs=[
            pl.BlockSpec((NC, BLK, D_OUT), lambda i: (0, i, 0)),
            pl.BlockSpec((BLK, D_OUT), lambda i: (i, 0)),
            pl.BlockSpec((BLK, 8), lambda i: (i, 0)),
        ],
        out_specs=pl.BlockSpec((BLK, D_OUT), lambda i: (i, 0)),
        out_shape=jax.ShapeDtypeStruct((N, D_OUT), jnp.float32),
    )(p2, v2, invd)


def kernel(features, edge_index, W1_self, W1_neigh, b1, W2_self, W2_neigh, b2):
    edges = edge_index.reshape(2, NCHUNK, CHUNK)
    feat_aug = jnp.concatenate(
        [features, jnp.ones((N, AUG - D_IN), jnp.float32)], axis=1)
    zeros_aug = jnp.zeros((N_PAD, AUG), jnp.float32)
    zeros_out = jnp.zeros((N_PAD, D_OUT), jnp.float32)

    p1 = _sc_agg_aug(feat_aug, edges, zeros_aug)
    u2, v2, invd = _tc1(p1, features, W1_self, W1_neigh,
                        b1.reshape(1, D_HID), W2_self, W2_neigh,
                        b2.reshape(1, D_OUT))
    p2 = _sc_agg_128(u2, edges, zeros_out)
    return _tc2(p2, v2, invd)
